# Initial kernel scaffold; baseline (speedup 1.0000x reference)
import jax
import jax.numpy as jnp
from jax.experimental import pallas as pl


def kernel(t, control_points, joint_points):
    raise NotImplementedError("write your pallas kernel here")


# speedup vs baseline: 13.9671x; 13.9671x over previous
import functools

import jax
import jax.numpy as jnp
from jax import lax
from jax.experimental import pallas as pl
from jax.experimental.pallas import tpu as pltpu
from jax.experimental.pallas import tpu_sc as plsc

DEGREE = 3
EPS = 1e-10
LANES = 16


@functools.cache
def _spline_sc(num_segments, n_t):
    info = plsc.get_sparse_core_info()
    nc, ns = info.num_cores, info.num_subcores
    nw = nc * ns
    chunk = n_t // nw
    n_rows = 2 * (DEGREE + 1)

    @functools.partial(
        pl.kernel,
        out_type=jax.ShapeDtypeStruct((2 * n_t,), jnp.float32),
        mesh=plsc.VectorSubcoreMesh(core_axis_name="c", subcore_axis_name="s"),
        compiler_params=pltpu.CompilerParams(needs_layout_passes=False),
        scratch_types=[
            pltpu.VMEM((chunk,), jnp.float32),
            pltpu.VMEM((2 * chunk,), jnp.float32),
            pltpu.VMEM((n_rows * num_segments,), jnp.float32),
        ],
    )
    def k(t_hbm, tab_hbm, out_hbm, t_v, out_v, tab_v):
        wid = lax.axis_index("s") * nc + lax.axis_index("c")
        base = wid * chunk
        pltpu.sync_copy(tab_hbm, tab_v)
        pltpu.sync_copy(t_hbm.at[pl.ds(base, chunk)], t_v)
        iv2 = lax.iota(jnp.int32, LANES) * 2
        nseg_f = float(num_segments)

        def body(i, carry):
            o = i * LANES
            tv = t_v[pl.ds(o, LANES)]
            u = jnp.minimum(tv, 1.0 - EPS) * nseg_f
            seg = u.astype(jnp.int32)
            pt = u - seg.astype(jnp.float32)
            is_last = seg == num_segments
            seg_c = jnp.where(is_last, num_segments - 1, seg)
            pt_c = jnp.where(is_last, 1.0, pt)
            omt = 1.0 - pt_c
            b0 = omt * omt * omt
            b1 = 3.0 * pt_c * omt * omt
            b2 = 3.0 * pt_c * pt_c * omt
            b3 = pt_c * pt_c * pt_c

            def g(row):
                return plsc.load_gather(tab_v, [seg_c + row * num_segments])

            x = b0 * g(0) + b1 * g(2) + b2 * g(4) + b3 * g(6)
            y = b0 * g(1) + b1 * g(3) + b2 * g(5) + b3 * g(7)
            oidx = iv2 + 2 * o
            plsc.store_scatter(out_v, [oidx], x)
            plsc.store_scatter(out_v, [oidx + 1], y)
            return carry

        lax.fori_loop(0, chunk // LANES, body, 0)
        pltpu.sync_copy(out_v, out_hbm.at[pl.ds(2 * base, 2 * chunk)])

    return k


def kernel(t, control_points, joint_points):
    ns = joint_points.shape[0] - 1
    n_t = t.shape[0]
    ctrl = control_points.reshape(ns, DEGREE - 1, 2)
    points = jnp.concatenate(
        [joint_points[:-1, None, :], ctrl, joint_points[1:, None, :]], axis=1
    )
    tab = points.transpose(1, 2, 0).reshape(2 * (DEGREE + 1) * ns)
    out = _spline_sc(ns, n_t)(t, tab)
    return out.reshape(n_t, 2)


# speedup vs baseline: 222.1641x; 15.9062x over previous
import functools

import jax
import jax.numpy as jnp
from jax import lax
from jax.experimental import pallas as pl
from jax.experimental.pallas import tpu as pltpu
from jax.experimental.pallas import tpu_sc as plsc

DEGREE = 3
EPS = 1e-10
LANES = 16


@functools.cache
def _spline_sc(num_segments, n_t):
    info = plsc.get_sparse_core_info()
    nc, ns = info.num_cores, info.num_subcores
    nw = nc * ns
    chunk = n_t // nw
    n_rows = 2 * (DEGREE + 1)

    @functools.partial(
        pl.kernel,
        out_type=jax.ShapeDtypeStruct((2, n_t), jnp.float32),
        mesh=plsc.VectorSubcoreMesh(core_axis_name="c", subcore_axis_name="s"),
        compiler_params=pltpu.CompilerParams(needs_layout_passes=False),
        scratch_types=[
            pltpu.VMEM((chunk,), jnp.float32),
            pltpu.VMEM((chunk,), jnp.float32),
            pltpu.VMEM((chunk,), jnp.float32),
            pltpu.VMEM((n_rows * num_segments,), jnp.float32),
        ],
    )
    def k(t_hbm, tab_hbm, out_hbm, t_v, x_v, y_v, tab_v):
        wid = lax.axis_index("s") * nc + lax.axis_index("c")
        base = wid * chunk
        pltpu.sync_copy(tab_hbm, tab_v)
        pltpu.sync_copy(t_hbm.at[pl.ds(base, chunk)], t_v)
        nseg_f = float(num_segments)

        def body(i, carry):
            o = i * LANES
            tv = t_v[pl.ds(o, LANES)]
            u = jnp.minimum(tv, 1.0 - EPS) * nseg_f
            seg = u.astype(jnp.int32)
            pt = u - seg.astype(jnp.float32)
            is_last = seg == num_segments
            seg_c = jnp.where(is_last, num_segments - 1, seg)
            pt_c = jnp.where(is_last, 1.0, pt)
            omt = 1.0 - pt_c
            b0 = omt * omt * omt
            b1 = 3.0 * pt_c * omt * omt
            b2 = 3.0 * pt_c * pt_c * omt
            b3 = pt_c * pt_c * pt_c

            def g(row):
                return plsc.load_gather(tab_v, [seg_c + row * num_segments])

            x = b0 * g(0) + b1 * g(2) + b2 * g(4) + b3 * g(6)
            y = b0 * g(1) + b1 * g(3) + b2 * g(5) + b3 * g(7)
            x_v[pl.ds(o, LANES)] = x
            y_v[pl.ds(o, LANES)] = y
            return carry

        lax.fori_loop(0, chunk // LANES, body, 0)
        pltpu.sync_copy(x_v, out_hbm.at[0, pl.ds(base, chunk)])
        pltpu.sync_copy(y_v, out_hbm.at[1, pl.ds(base, chunk)])

    return k


def kernel(t, control_points, joint_points):
    ns = joint_points.shape[0] - 1
    n_t = t.shape[0]
    ctrl = control_points.reshape(ns, DEGREE - 1, 2)
    points = jnp.concatenate(
        [joint_points[:-1, None, :], ctrl, joint_points[1:, None, :]], axis=1
    )
    tab = points.transpose(1, 2, 0).reshape(2 * (DEGREE + 1) * ns)
    out = _spline_sc(ns, n_t)(t, tab)
    return out.T


# speedup vs baseline: 249.6274x; 1.1236x over previous
import functools

import jax
import jax.numpy as jnp
import numpy as np
from jax import lax
from jax.experimental import pallas as pl
from jax.experimental.pallas import tpu as pltpu
from jax.experimental.pallas import tpu_sc as plsc

DEGREE = 3
EPS = 1e-10
LANES = 16


@functools.cache
def _spline_sc(num_segments, n_t):
    info = plsc.get_sparse_core_info()
    nc, ns = info.num_cores, info.num_subcores
    nw = nc * ns
    chunk = n_t // nw
    n_rows = 2 * (DEGREE + 1)

    @functools.partial(
        pl.kernel,
        out_type=jax.ShapeDtypeStruct((2, n_t), jnp.float32),
        mesh=plsc.VectorSubcoreMesh(core_axis_name="c", subcore_axis_name="s"),
        compiler_params=pltpu.CompilerParams(needs_layout_passes=False),
        scratch_types=[
            pltpu.VMEM((chunk,), jnp.float32),
            pltpu.VMEM((chunk,), jnp.float32),
            pltpu.VMEM((chunk,), jnp.float32),
            pltpu.VMEM((n_rows * num_segments,), jnp.float32),
        ],
    )
    def k(t_hbm, tab_hbm, out_hbm, t_v, x_v, y_v, tab_v):
        wid = lax.axis_index("s") * nc + lax.axis_index("c")
        base = wid * chunk
        pltpu.sync_copy(tab_hbm, tab_v)
        pltpu.sync_copy(t_hbm.at[pl.ds(base, chunk)], t_v)
        nseg_f = float(num_segments)
        tmax = float(np.nextafter(np.float32(1.0), np.float32(0.0)))

        @plsc.parallel_loop(0, chunk, LANES, unroll=4)
        def body(o):
            tv = t_v[pl.ds(o, LANES)]
            u = jnp.minimum(tv, tmax) * nseg_f
            seg = u.astype(jnp.int32)
            pt = u - seg.astype(jnp.float32)
            seg8 = seg << 3
            omt = 1.0 - pt
            o2 = omt * omt
            p2 = pt * pt
            p3t = 3.0 * pt
            b0 = o2 * omt
            b1 = p3t * o2
            b2 = 3.0 * p2 * omt
            b3 = p2 * pt

            def g(row):
                return plsc.load_gather(tab_v, [seg8 + row])

            x = b0 * g(0) + b1 * g(2) + b2 * g(4) + b3 * g(6)
            y = b0 * g(1) + b1 * g(3) + b2 * g(5) + b3 * g(7)
            x_v[pl.ds(o, LANES)] = x
            y_v[pl.ds(o, LANES)] = y

        pltpu.sync_copy(x_v, out_hbm.at[0, pl.ds(base, chunk)])
        pltpu.sync_copy(y_v, out_hbm.at[1, pl.ds(base, chunk)])

    return k


def kernel(t, control_points, joint_points):
    ns = joint_points.shape[0] - 1
    n_t = t.shape[0]
    ctrl = control_points.reshape(ns, DEGREE - 1, 2)
    points = jnp.concatenate(
        [joint_points[:-1, None, :], ctrl, joint_points[1:, None, :]], axis=1
    )
    tab = points.reshape(2 * (DEGREE + 1) * ns)
    out = _spline_sc(ns, n_t)(t, tab)
    return out.T


# speedup vs baseline: 252.9117x; 1.0132x over previous
import functools

import jax
import jax.numpy as jnp
import numpy as np
from jax import lax
from jax.experimental import pallas as pl
from jax.experimental.pallas import tpu as pltpu
from jax.experimental.pallas import tpu_sc as plsc

DEGREE = 3
LANES = 16


@functools.cache
def _spline_sc(num_segments, n_t):
    info = plsc.get_sparse_core_info()
    nc, ns = info.num_cores, info.num_subcores
    nw = nc * ns
    chunk = n_t // nw
    n_cp = num_segments * (DEGREE - 1) * 2
    n_jp = (num_segments + 1) * 2

    @functools.partial(
        pl.kernel,
        out_type=jax.ShapeDtypeStruct((2, n_t), jnp.float32),
        mesh=plsc.VectorSubcoreMesh(core_axis_name="c", subcore_axis_name="s"),
        compiler_params=pltpu.CompilerParams(needs_layout_passes=False),
        scratch_types=[
            pltpu.VMEM((chunk,), jnp.float32),
            pltpu.VMEM((chunk,), jnp.float32),
            pltpu.VMEM((chunk,), jnp.float32),
            pltpu.VMEM((8 * num_segments,), jnp.float32),
            pltpu.VMEM((n_cp,), jnp.float32),
            pltpu.VMEM((n_jp,), jnp.float32),
        ],
    )
    def k(t_hbm, cp_hbm, jp_hbm, out_hbm, t_v, x_v, y_v, tab_v, cp_v, jp_v):
        wid = lax.axis_index("s") * nc + lax.axis_index("c")
        base = wid * chunk
        pltpu.sync_copy(cp_hbm, cp_v)
        pltpu.sync_copy(jp_hbm, jp_v)
        pltpu.sync_copy(t_hbm.at[pl.ds(base, chunk)], t_v)
        iv = lax.iota(jnp.int32, LANES)
        nseg_f = float(num_segments)
        tmax = float(np.nextafter(np.float32(1.0), np.float32(0.0)))

        @plsc.parallel_loop(0, num_segments, LANES)
        def build(s0):
            segv = s0 + iv
            j2 = segv * 2
            c4 = segv * 4
            s8 = segv * 8

            def jp(off):
                return plsc.load_gather(jp_v, [j2 + off])

            def cp(off):
                return plsc.load_gather(cp_v, [c4 + off])

            plsc.store_scatter(tab_v, [s8], jp(0))
            plsc.store_scatter(tab_v, [s8 + 1], jp(1))
            plsc.store_scatter(tab_v, [s8 + 2], cp(0))
            plsc.store_scatter(tab_v, [s8 + 3], cp(1))
            plsc.store_scatter(tab_v, [s8 + 4], cp(2))
            plsc.store_scatter(tab_v, [s8 + 5], cp(3))
            plsc.store_scatter(tab_v, [s8 + 6], jp(2))
            plsc.store_scatter(tab_v, [s8 + 7], jp(3))

        @plsc.parallel_loop(0, chunk, LANES, unroll=4)
        def body(o):
            tv = t_v[pl.ds(o, LANES)]
            u = jnp.minimum(tv, tmax) * nseg_f
            seg = u.astype(jnp.int32)
            pt = u - seg.astype(jnp.float32)
            seg8 = seg << 3
            omt = 1.0 - pt
            o2 = omt * omt
            p2 = pt * pt
            p3t = 3.0 * pt
            b0 = o2 * omt
            b1 = p3t * o2
            b2 = 3.0 * p2 * omt
            b3 = p2 * pt

            def g(row):
                return plsc.load_gather(tab_v, [seg8 + row])

            x = b0 * g(0) + b1 * g(2) + b2 * g(4) + b3 * g(6)
            y = b0 * g(1) + b1 * g(3) + b2 * g(5) + b3 * g(7)
            x_v[pl.ds(o, LANES)] = x
            y_v[pl.ds(o, LANES)] = y

        pltpu.sync_copy(x_v, out_hbm.at[0, pl.ds(base, chunk)])
        pltpu.sync_copy(y_v, out_hbm.at[1, pl.ds(base, chunk)])

    return k


def kernel(t, control_points, joint_points):
    ns = joint_points.shape[0] - 1
    n_t = t.shape[0]
    out = _spline_sc(ns, n_t)(
        t, control_points.reshape(-1), joint_points.reshape(-1)
    )
    return out.T


# speedup vs baseline: 273.7074x; 1.0822x over previous
import functools

import jax
import jax.numpy as jnp
import numpy as np
from jax import lax
from jax.experimental import pallas as pl
from jax.experimental.pallas import tpu as pltpu
from jax.experimental.pallas import tpu_sc as plsc

DEGREE = 3
LANES = 16


@functools.cache
def _spline_sc(num_segments, n_t):
    info = plsc.get_sparse_core_info()
    nc, ns = info.num_cores, info.num_subcores
    nw = nc * ns
    chunk = n_t // nw
    NB = 4
    SUB = chunk // NB
    n_cp = num_segments * (DEGREE - 1) * 2
    n_jp = (num_segments + 1) * 2

    @functools.partial(
        pl.kernel,
        out_type=jax.ShapeDtypeStruct((2, n_t), jnp.float32),
        mesh=plsc.VectorSubcoreMesh(core_axis_name="c", subcore_axis_name="s"),
        compiler_params=pltpu.CompilerParams(needs_layout_passes=False),
        scratch_types=[
            pltpu.VMEM((SUB,), jnp.float32),
            pltpu.VMEM((SUB,), jnp.float32),
            pltpu.VMEM((SUB,), jnp.float32),
            pltpu.VMEM((SUB,), jnp.float32),
            pltpu.VMEM((SUB,), jnp.float32),
            pltpu.VMEM((SUB,), jnp.float32),
            pltpu.VMEM((8 * num_segments,), jnp.float32),
            pltpu.VMEM((n_cp,), jnp.float32),
            pltpu.VMEM((n_jp,), jnp.float32),
            pltpu.SemaphoreType.DMA,
            pltpu.SemaphoreType.DMA,
            pltpu.SemaphoreType.DMA,
            pltpu.SemaphoreType.DMA,
        ],
    )
    def k(t_hbm, cp_hbm, jp_hbm, out_hbm,
          t0_v, t1_v, x0_v, x1_v, y0_v, y1_v, tab_v, cp_v, jp_v,
          in0_s, in1_s, out0_s, out1_s):
        wid = lax.axis_index("s") * nc + lax.axis_index("c")
        base = wid * chunk
        t_bufs = (t0_v, t1_v)
        x_bufs = (x0_v, x1_v)
        y_bufs = (y0_v, y1_v)
        in_sems = (in0_s, in1_s)
        out_sems = (out0_s, out1_s)

        def in_copy(j, b):
            return pltpu.make_async_copy(
                t_hbm.at[pl.ds(base + j * SUB, SUB)], t_bufs[b], in_sems[b]
            )

        def out_copy(j, b, row, buf):
            return pltpu.make_async_copy(
                buf, out_hbm.at[row, pl.ds(base + j * SUB, SUB)], out_sems[b]
            )

        in_copy(0, 0).start()
        pltpu.sync_copy(cp_hbm, cp_v)
        pltpu.sync_copy(jp_hbm, jp_v)
        iv = lax.iota(jnp.int32, LANES)
        nseg_f = float(num_segments)
        tmax = float(np.nextafter(np.float32(1.0), np.float32(0.0)))

        @plsc.parallel_loop(0, num_segments, LANES)
        def build(s0):
            segv = s0 + iv
            j2 = segv * 2
            c4 = segv * 4
            s8 = segv * 8

            def jp(off):
                return plsc.load_gather(jp_v, [j2 + off])

            def cp(off):
                return plsc.load_gather(cp_v, [c4 + off])

            plsc.store_scatter(tab_v, [s8], jp(0))
            plsc.store_scatter(tab_v, [s8 + 1], jp(1))
            plsc.store_scatter(tab_v, [s8 + 2], cp(0))
            plsc.store_scatter(tab_v, [s8 + 3], cp(1))
            plsc.store_scatter(tab_v, [s8 + 4], cp(2))
            plsc.store_scatter(tab_v, [s8 + 5], cp(3))
            plsc.store_scatter(tab_v, [s8 + 6], jp(2))
            plsc.store_scatter(tab_v, [s8 + 7], jp(3))

        def compute(t_v, x_v, y_v):
            @plsc.parallel_loop(0, SUB, LANES, unroll=4)
            def body(o):
                tv = t_v[pl.ds(o, LANES)]
                u = jnp.minimum(tv, tmax) * nseg_f
                seg = u.astype(jnp.int32)
                pt = u - seg.astype(jnp.float32)
                seg8 = seg << 3
                omt = 1.0 - pt
                o2 = omt * omt
                p2 = pt * pt
                p3t = 3.0 * pt
                b0 = o2 * omt
                b1 = p3t * o2
                b2 = 3.0 * p2 * omt
                b3 = p2 * pt

                def g(row):
                    return plsc.load_gather(tab_v, [seg8 + row])

                x = b0 * g(0) + b1 * g(2) + b2 * g(4) + b3 * g(6)
                y = b0 * g(1) + b1 * g(3) + b2 * g(5) + b3 * g(7)
                x_v[pl.ds(o, LANES)] = x
                y_v[pl.ds(o, LANES)] = y

        for j in range(NB):
            b = j % 2
            in_copy(j, b).wait()
            if j + 1 < NB:
                in_copy(j + 1, 1 - b).start()
            if j >= 2:
                out_copy(j - 2, b, 0, x_bufs[b]).wait()
                out_copy(j - 2, b, 1, y_bufs[b]).wait()
            compute(t_bufs[b], x_bufs[b], y_bufs[b])
            out_copy(j, b, 0, x_bufs[b]).start()
            out_copy(j, b, 1, y_bufs[b]).start()
        for j in (NB - 2, NB - 1):
            b = j % 2
            out_copy(j, b, 0, x_bufs[b]).wait()
            out_copy(j, b, 1, y_bufs[b]).wait()

    return k


def kernel(t, control_points, joint_points):
    ns = joint_points.shape[0] - 1
    n_t = t.shape[0]
    out = _spline_sc(ns, n_t)(
        t, control_points.reshape(-1), joint_points.reshape(-1)
    )
    return out.T
